# baseline (speedup 1.0000x reference)
import jax
import jax.numpy as jnp
from jax import lax
from jax.experimental import pallas as pl
from jax.experimental.pallas import tpu as pltpu
from jax.experimental.pallas import tpu_sc as plsc

N_NODES = 10000
N_EDGES = 320000
D = 128

NC = 2
NS = 16
NW = NC * NS
E_PER_W = N_EDGES // NW
CHUNK = 80
NPROC = 128
NIDX = NPROC + 4
NRING = 8
ROWS_PER_TILE = 624
TAIL_ROWS = N_NODES - NS * ROWS_PER_TILE
ZROWS = 16
LANES = 16


def _mm_body(x_ref, w_ref, o_ref):
    o_ref[...] = jnp.dot(x_ref[...], w_ref[...], preferred_element_type=jnp.float32)


def _matmul(x, w):
    m_blk = 2000
    return pl.pallas_call(
        _mm_body,
        out_shape=jax.ShapeDtypeStruct((N_NODES, D), jnp.float32),
        grid=(N_NODES // m_blk,),
        in_specs=[
            pl.BlockSpec((m_blk, D), lambda i: (i, 0)),
            pl.BlockSpec((D, D), lambda i: (0, 0)),
        ],
        out_specs=pl.BlockSpec((m_blk, D), lambda i: (i, 0)),
    )(x, w)


def _sum_mm_body(acc_ref, b_ref, w_ref, o_ref):
    x = acc_ref[0] + acc_ref[1] + b_ref[...]
    o_ref[...] = jnp.dot(x, w_ref[...], preferred_element_type=jnp.float32)


def _sum_matmul(acc, b, w):
    m_blk = 2000
    return pl.pallas_call(
        _sum_mm_body,
        out_shape=jax.ShapeDtypeStruct((N_NODES, D), jnp.float32),
        grid=(N_NODES // m_blk,),
        in_specs=[
            pl.BlockSpec((2, m_blk, D), lambda i: (0, i, 0)),
            pl.BlockSpec((D,), lambda i: (0,)),
            pl.BlockSpec((D, D), lambda i: (0, 0)),
        ],
        out_specs=pl.BlockSpec((m_blk, D), lambda i: (i, 0)),
    )(acc, b, w)


def _sum_bias_body(acc_ref, b_ref, o_ref):
    o_ref[...] = acc_ref[0] + acc_ref[1] + b_ref[...]


def _sum_bias(acc, b):
    m_blk = 2000
    return pl.pallas_call(
        _sum_bias_body,
        out_shape=jax.ShapeDtypeStruct((N_NODES, D), jnp.float32),
        grid=(N_NODES // m_blk,),
        in_specs=[
            pl.BlockSpec((2, m_blk, D), lambda i: (0, i, 0)),
            pl.BlockSpec((D,), lambda i: (0,)),
        ],
        out_specs=pl.BlockSpec((m_blk, D), lambda i: (i, 0)),
    )(acc, b)


def _edge_body(h_hbm, src_hbm, dst_hbm, w_hbm, out_hbm,
               src_r, dst_r, w_r,
               gbuf0, gbuf1, sbuf0, sbuf1, zbuf, acc_sh,
               sg0, sg1, ss0, ss1, si0, si1):
    cid = lax.axis_index("c")
    sid = lax.axis_index("s")
    wid = sid * NC + cid

    zero = jnp.zeros((LANES,), jnp.float32)

    def zrow(i, carry):
        for v in range(D // LANES):
            zbuf[i, pl.ds(v * LANES, LANES)] = zero
        return carry

    lax.fori_loop(0, ZROWS, zrow, 0)

    def zcopy(k, carry):
        pltpu.sync_copy(zbuf,
                        acc_sh.at[pl.ds(sid * ROWS_PER_TILE + k * ZROWS, ZROWS)])
        return carry

    lax.fori_loop(0, ROWS_PER_TILE // ZROWS, zcopy, 0)

    @pl.when(sid == NS - 1)
    def _zero_tail():
        pltpu.sync_copy(zbuf.at[pl.ds(0, TAIL_ROWS)],
                        acc_sh.at[pl.ds(NS * ROWS_PER_TILE, TAIL_ROWS)])

    for j in range(4):
        pltpu.sync_copy(src_hbm.at[wid, j], src_r.at[j])
        pltpu.sync_copy(dst_hbm.at[wid, j], dst_r.at[j])
        pltpu.sync_copy(w_hbm.at[wid, j], w_r.at[j])

    plsc.subcore_barrier()

    def fire_g(j, gbuf, sem):
        pltpu.async_copy(h_hbm.at[src_r.at[lax.rem(j, NRING)]], gbuf, sem)

    def wait_g(j, gbuf, sem):
        pltpu.make_async_copy(h_hbm.at[src_r.at[lax.rem(j, NRING)]], gbuf,
                              sem).wait()

    def fire_idx(j, sem):
        slot = lax.rem(j, NRING)
        pltpu.async_copy(src_hbm.at[wid, j], src_r.at[slot], sem)
        pltpu.async_copy(dst_hbm.at[wid, j], dst_r.at[slot], sem)
        pltpu.async_copy(w_hbm.at[wid, j], w_r.at[slot], sem)

    def wait_idx(j, sem):
        slot = lax.rem(j, NRING)
        pltpu.make_async_copy(src_hbm.at[wid, j], src_r.at[slot], sem).wait()
        pltpu.make_async_copy(dst_hbm.at[wid, j], dst_r.at[slot], sem).wait()
        pltpu.make_async_copy(w_hbm.at[wid, j], w_r.at[slot], sem).wait()

    def fire_s(j, sbuf, sem):
        pltpu.async_copy(sbuf, acc_sh.at[dst_r.at[lax.rem(j, NRING)]], sem,
                         add=True)

    def wait_s(j, sbuf, sem):
        pltpu.make_async_copy(sbuf, acc_sh.at[dst_r.at[lax.rem(j, NRING)]],
                              sem).wait()

    def scale(j, gbuf, sbuf):
        slot = lax.rem(j, NRING)

        def body(g, carry):
            for l in range(16):
                e = g * 16 + l
                wspl = plsc.load_gather(
                    w_r, [jnp.full((LANES,), slot, jnp.int32),
                          jnp.full((LANES,), e, jnp.int32)])
                for v in range(D // LANES):
                    sbuf[e, pl.ds(v * LANES, LANES)] = (
                        gbuf[e, pl.ds(v * LANES, LANES)] * wspl)
            return carry

        lax.fori_loop(0, CHUNK // 16, body, 0)

    fire_g(0, gbuf0, sg0)
    fire_g(1, gbuf1, sg1)

    wait_g(0, gbuf0, sg0)
    scale(0, gbuf0, sbuf0)
    fire_g(2, gbuf0, sg0)
    fire_idx(4, si0)
    fire_s(0, sbuf0, ss0)

    wait_g(1, gbuf1, sg1)
    scale(1, gbuf1, sbuf1)
    fire_g(3, gbuf1, sg1)
    fire_idx(5, si1)
    fire_s(1, sbuf1, ss1)

    def loop_body(jj, carry):
        j0 = 2 * jj
        j1 = j0 + 1

        wait_g(j0, gbuf0, sg0)
        wait_s(j0 - 2, sbuf0, ss0)
        scale(j0, gbuf0, sbuf0)
        wait_idx(j0 + 2, si0)
        fire_g(j0 + 2, gbuf0, sg0)
        fire_idx(j0 + 4, si0)
        fire_s(j0, sbuf0, ss0)

        wait_g(j1, gbuf1, sg1)
        wait_s(j1 - 2, sbuf1, ss1)
        scale(j1, gbuf1, sbuf1)
        wait_idx(j1 + 2, si1)
        fire_g(j1 + 2, gbuf1, sg1)
        fire_idx(j1 + 4, si1)
        fire_s(j1, sbuf1, ss1)
        return carry

    lax.fori_loop(1, NPROC // 2, loop_body, 0)

    wait_g(NPROC, gbuf0, sg0)
    wait_g(NPROC + 1, gbuf1, sg1)
    wait_idx(NPROC + 2, si0)
    wait_idx(NPROC + 3, si1)
    wait_s(NPROC - 2, sbuf0, ss0)
    wait_s(NPROC - 1, sbuf1, ss1)

    plsc.subcore_barrier()

    pltpu.sync_copy(acc_sh.at[pl.ds(sid * ROWS_PER_TILE, ROWS_PER_TILE)],
                    out_hbm.at[cid, pl.ds(sid * ROWS_PER_TILE, ROWS_PER_TILE)])

    @pl.when(sid == NS - 1)
    def _write_tail():
        pltpu.sync_copy(acc_sh.at[pl.ds(NS * ROWS_PER_TILE, TAIL_ROWS)],
                        out_hbm.at[cid, pl.ds(NS * ROWS_PER_TILE, TAIL_ROWS)])


def _edge_phase(h, src3, dst3, ew3):
    mesh = plsc.VectorSubcoreMesh(core_axis_name="c", subcore_axis_name="s")
    f = pl.kernel(
        _edge_body,
        out_type=jax.ShapeDtypeStruct((NC, N_NODES, D), jnp.float32),
        mesh=mesh,
        scratch_types=[
            pltpu.VMEM((NRING, CHUNK), jnp.int32),
            pltpu.VMEM((NRING, CHUNK), jnp.int32),
            pltpu.VMEM((NRING, CHUNK), jnp.float32),
            pltpu.VMEM((CHUNK, D), jnp.float32),
            pltpu.VMEM((CHUNK, D), jnp.float32),
            pltpu.VMEM((CHUNK, D), jnp.float32),
            pltpu.VMEM((CHUNK, D), jnp.float32),
            pltpu.VMEM((ZROWS, D), jnp.float32),
            pltpu.VMEM_SHARED((N_NODES, D), jnp.float32),
            pltpu.SemaphoreType.DMA,
            pltpu.SemaphoreType.DMA,
            pltpu.SemaphoreType.DMA,
            pltpu.SemaphoreType.DMA,
            pltpu.SemaphoreType.DMA,
            pltpu.SemaphoreType.DMA,
        ],
        compiler_params=pltpu.CompilerParams(needs_layout_passes=False),
    )
    return f(h, src3, dst3, ew3)


def kernel(features, edge_index, edge_weight, W1, b1, W2, b2):
    pad = NIDX * CHUNK - E_PER_W
    src3 = jnp.pad(edge_index[0].reshape(NW, E_PER_W),
                   ((0, 0), (0, pad))).reshape(NW, NIDX, CHUNK)
    dst3 = jnp.pad(edge_index[1].reshape(NW, E_PER_W),
                   ((0, 0), (0, pad))).reshape(NW, NIDX, CHUNK)
    ew3 = jnp.pad(edge_weight.reshape(NW, E_PER_W),
                  ((0, 0), (0, pad))).reshape(NW, NIDX, CHUNK)

    h1 = _matmul(features, W1)
    acc1 = _edge_phase(h1, src3, dst3, ew3)
    h2 = _sum_matmul(acc1, b1, W2)
    acc2 = _edge_phase(h2, src3, dst3, ew3)
    return _sum_bias(acc2, b2)


# speedup vs baseline: 2.1105x; 2.1105x over previous
import jax
import jax.numpy as jnp
from jax import lax
from jax.experimental import pallas as pl
from jax.experimental.pallas import tpu as pltpu
from jax.experimental.pallas import tpu_sc as plsc

N_NODES = 10000
N_EDGES = 320000
D = 128

NC = 2
NS = 16
NW = NC * NS
E_PER_W = N_EDGES // NW
CHUNK = 64
NPROC = 158
NIDX = NPROC
NRING = 4
ROWS_PER_TILE = 624
TAIL_ROWS = N_NODES - NS * ROWS_PER_TILE
ZROWS = 16
LANES = 16


def _mm_body(x_ref, w_ref, o_ref):
    o_ref[...] = jnp.dot(x_ref[...], w_ref[...], preferred_element_type=jnp.float32)


def _matmul(x, w):
    m_blk = 2000
    return pl.pallas_call(
        _mm_body,
        out_shape=jax.ShapeDtypeStruct((N_NODES, D), jnp.float32),
        grid=(N_NODES // m_blk,),
        in_specs=[
            pl.BlockSpec((m_blk, D), lambda i: (i, 0)),
            pl.BlockSpec((D, D), lambda i: (0, 0)),
        ],
        out_specs=pl.BlockSpec((m_blk, D), lambda i: (i, 0)),
    )(x, w)


def _sum_mm_body(acc_ref, b_ref, w_ref, o_ref):
    x = acc_ref[0] + acc_ref[1] + b_ref[...]
    o_ref[...] = jnp.dot(x, w_ref[...], preferred_element_type=jnp.float32)


def _sum_matmul(acc, b, w):
    m_blk = 2000
    return pl.pallas_call(
        _sum_mm_body,
        out_shape=jax.ShapeDtypeStruct((N_NODES, D), jnp.float32),
        grid=(N_NODES // m_blk,),
        in_specs=[
            pl.BlockSpec((2, m_blk, D), lambda i: (0, i, 0)),
            pl.BlockSpec((D,), lambda i: (0,)),
            pl.BlockSpec((D, D), lambda i: (0, 0)),
        ],
        out_specs=pl.BlockSpec((m_blk, D), lambda i: (i, 0)),
    )(acc, b, w)


def _sum_bias_body(acc_ref, b_ref, o_ref):
    o_ref[...] = acc_ref[0] + acc_ref[1] + b_ref[...]


def _sum_bias(acc, b):
    m_blk = 2000
    return pl.pallas_call(
        _sum_bias_body,
        out_shape=jax.ShapeDtypeStruct((N_NODES, D), jnp.float32),
        grid=(N_NODES // m_blk,),
        in_specs=[
            pl.BlockSpec((2, m_blk, D), lambda i: (0, i, 0)),
            pl.BlockSpec((D,), lambda i: (0,)),
        ],
        out_specs=pl.BlockSpec((m_blk, D), lambda i: (i, 0)),
    )(acc, b)


def _edge_body(h_hbm, src_hbm, dst_hbm, w_hbm, out_hbm,
               src_all, dst_r, w_r,
               gbuf0, gbuf1, sbuf0, sbuf1, zbuf, acc_sh,
               sg0, sg1, ss0, ss1):
    cid = lax.axis_index("c")
    sid = lax.axis_index("s")
    wid = sid * NC + cid

    zero = jnp.zeros((LANES,), jnp.float32)

    def zrow(i, carry):
        for v in range(D // LANES):
            zbuf[i, pl.ds(v * LANES, LANES)] = zero
        return carry

    lax.fori_loop(0, ZROWS, zrow, 0)

    def zcopy(k, carry):
        pltpu.sync_copy(zbuf,
                        acc_sh.at[pl.ds(sid * ROWS_PER_TILE + k * ZROWS, ZROWS)])
        return carry

    lax.fori_loop(0, ROWS_PER_TILE // ZROWS, zcopy, 0)

    @pl.when(sid == NS - 1)
    def _zero_tail():
        pltpu.sync_copy(zbuf.at[pl.ds(0, TAIL_ROWS)],
                        acc_sh.at[pl.ds(NS * ROWS_PER_TILE, TAIL_ROWS)])

    pltpu.sync_copy(src_hbm.at[wid], src_all)

    plsc.subcore_barrier()

    def fire_g(j, gbuf, sem):
        slot = lax.rem(j, NRING)
        idx = src_all.at[pl.ds(j * CHUNK, CHUNK)]
        pltpu.async_copy(h_hbm.at[idx], gbuf, sem)
        pltpu.async_copy(dst_hbm.at[wid, j], dst_r.at[slot], sem)
        pltpu.async_copy(w_hbm.at[wid, j], w_r.at[slot], sem)

    def wait_g(j, gbuf, sem):
        slot = lax.rem(j, NRING)
        idx = src_all.at[pl.ds(j * CHUNK, CHUNK)]
        pltpu.make_async_copy(h_hbm.at[idx], gbuf, sem).wait()
        pltpu.make_async_copy(dst_hbm.at[wid, j], dst_r.at[slot], sem).wait()
        pltpu.make_async_copy(w_hbm.at[wid, j], w_r.at[slot], sem).wait()

    def fire_s(j, sbuf, sem):
        pltpu.async_copy(sbuf, acc_sh.at[dst_r.at[lax.rem(j, NRING)]], sem,
                         add=True)

    def wait_s(j, sbuf, sem):
        pltpu.make_async_copy(sbuf, acc_sh.at[dst_r.at[lax.rem(j, NRING)]],
                              sem).wait()

    def scale(j, gbuf, sbuf):
        slot = lax.rem(j, NRING)

        def body(g, carry):
            for l in range(16):
                e = g * 16 + l
                wspl = plsc.load_gather(
                    w_r, [jnp.full((LANES,), slot, jnp.int32),
                          jnp.full((LANES,), e, jnp.int32)])
                for v in range(D // LANES):
                    sbuf[e, pl.ds(v * LANES, LANES)] = (
                        gbuf[e, pl.ds(v * LANES, LANES)] * wspl)
            return carry

        lax.fori_loop(0, CHUNK // 16, body, 0)

    fire_g(0, gbuf0, sg0)
    fire_g(1, gbuf1, sg1)

    wait_g(0, gbuf0, sg0)
    scale(0, gbuf0, sbuf0)
    fire_g(2, gbuf0, sg0)
    fire_s(0, sbuf0, ss0)

    wait_g(1, gbuf1, sg1)
    scale(1, gbuf1, sbuf1)
    fire_g(3, gbuf1, sg1)
    fire_s(1, sbuf1, ss1)

    def loop_body(jj, carry):
        j0 = 2 * jj
        j1 = j0 + 1

        wait_g(j0, gbuf0, sg0)
        wait_s(j0 - 2, sbuf0, ss0)
        scale(j0, gbuf0, sbuf0)
        fire_g(j0 + 2, gbuf0, sg0)
        fire_s(j0, sbuf0, ss0)

        wait_g(j1, gbuf1, sg1)
        wait_s(j1 - 2, sbuf1, ss1)
        scale(j1, gbuf1, sbuf1)
        fire_g(j1 + 2, gbuf1, sg1)
        fire_s(j1, sbuf1, ss1)
        return carry

    lax.fori_loop(1, NPROC // 2 - 1, loop_body, 0)

    jE = NPROC - 2
    wait_g(jE, gbuf0, sg0)
    wait_s(jE - 2, sbuf0, ss0)
    scale(jE, gbuf0, sbuf0)
    fire_s(jE, sbuf0, ss0)
    wait_g(jE + 1, gbuf1, sg1)
    wait_s(jE - 1, sbuf1, ss1)
    scale(jE + 1, gbuf1, sbuf1)
    fire_s(jE + 1, sbuf1, ss1)
    wait_s(jE, sbuf0, ss0)
    wait_s(jE + 1, sbuf1, ss1)

    plsc.subcore_barrier()

    pltpu.sync_copy(acc_sh.at[pl.ds(sid * ROWS_PER_TILE, ROWS_PER_TILE)],
                    out_hbm.at[cid, pl.ds(sid * ROWS_PER_TILE, ROWS_PER_TILE)])

    @pl.when(sid == NS - 1)
    def _write_tail():
        pltpu.sync_copy(acc_sh.at[pl.ds(NS * ROWS_PER_TILE, TAIL_ROWS)],
                        out_hbm.at[cid, pl.ds(NS * ROWS_PER_TILE, TAIL_ROWS)])


def _edge_phase(h, src3, dst3, ew3):
    mesh = plsc.VectorSubcoreMesh(core_axis_name="c", subcore_axis_name="s")
    f = pl.kernel(
        _edge_body,
        out_type=jax.ShapeDtypeStruct((NC, N_NODES, D), jnp.float32),
        mesh=mesh,
        scratch_types=[
            pltpu.VMEM((NIDX * CHUNK,), jnp.int32),
            pltpu.VMEM((NRING, CHUNK), jnp.int32),
            pltpu.VMEM((NRING, CHUNK), jnp.float32),
            pltpu.VMEM((CHUNK, D), jnp.float32),
            pltpu.VMEM((CHUNK, D), jnp.float32),
            pltpu.VMEM((CHUNK, D), jnp.float32),
            pltpu.VMEM((CHUNK, D), jnp.float32),
            pltpu.VMEM((ZROWS, D), jnp.float32),
            pltpu.VMEM_SHARED((N_NODES, D), jnp.float32),
            pltpu.SemaphoreType.DMA,
            pltpu.SemaphoreType.DMA,
            pltpu.SemaphoreType.DMA,
            pltpu.SemaphoreType.DMA,
        ],
        compiler_params=pltpu.CompilerParams(needs_layout_passes=False),
    )
    return f(h, src3, dst3, ew3)


def kernel(features, edge_index, edge_weight, W1, b1, W2, b2):
    pad = NIDX * CHUNK - E_PER_W
    src3 = jnp.pad(edge_index[0].reshape(NW, E_PER_W), ((0, 0), (0, pad)))
    dst3 = jnp.pad(edge_index[1].reshape(NW, E_PER_W),
                   ((0, 0), (0, pad))).reshape(NW, NIDX, CHUNK)
    ew3 = jnp.pad(edge_weight.reshape(NW, E_PER_W),
                  ((0, 0), (0, pad))).reshape(NW, NIDX, CHUNK)

    h1 = _matmul(features, W1)
    acc1 = _edge_phase(h1, src3, dst3, ew3)
    h2 = _sum_matmul(acc1, b1, W2)
    acc2 = _edge_phase(h2, src3, dst3, ew3)
    return _sum_bias(acc2, b2)
